# baseline (speedup 1.0000x reference)
import functools

import jax
import jax.numpy as jnp
from jax import lax
from jax.experimental import pallas as pl
from jax.experimental.pallas import tpu as pltpu
from jax.experimental.pallas import tpu_sc as plsc

B = 1024
L = 200
H = 128
POS_ROWS = 513
SEG_ROWS = 3
N = B * L
NW = 32
PER_W = N // NW
C = 128
NCHUNK = PER_W // C
NCOL = H // 16


def _tokpos_tc_body(tok_ref, pos_ref, out_ref):
    out_ref[...] = tok_ref[...] + pos_ref[...]


def _sc_body(x_hbm, xseg_hbm, tokpos_hbm, seg_hbm, out_hbm,
             xi, si, tp, sg, st0, st1, so0, so1):
    wid = lax.axis_index("s") * 2 + lax.axis_index("c")
    base = wid * PER_W
    pltpu.sync_copy(x_hbm.at[wid], xi)
    pltpu.sync_copy(xseg_hbm.at[wid], si)
    pltpu.sync_copy(tokpos_hbm, tp)
    pltpu.sync_copy(seg_hbm, sg)

    segv = [[sg[s, pl.ds(j * 16, 16)] for j in range(NCOL)]
            for s in range(SEG_ROWS)]
    stages = (st0, st1)
    sos = (so0, so1)

    def wait_out(b):
        pltpu.make_async_copy(
            stages[b], out_hbm.at[pl.ds(base, C)], sos[b]).wait()

    def expand_chunk(i, b):
        st = stages[b]

        def grp(g, carry):
            pvec = xi[i, pl.ds(g * 16, 16)]
            svec = si[i, pl.ds(g * 16, 16)]
            for l in range(16):
                p = pvec[l]
                s = svec[l]
                m1 = s == 1
                m2 = s == 2
                for j in range(NCOL):
                    tv = tp[p, pl.ds(j * 16, 16)]
                    sv = jnp.where(m2, segv[2][j],
                                   jnp.where(m1, segv[1][j], segv[0][j]))
                    st[g * 16 + l, pl.ds(j * 16, 16)] = tv + sv
            return carry

        lax.fori_loop(0, C // 16, grp, 0)

    def step(k, carry):
        for b in range(2):
            i = 2 * k + b

            @pl.when(k > 0)
            def _():
                wait_out(b)

            expand_chunk(i, b)
            pltpu.async_copy(
                stages[b], out_hbm.at[pl.ds(base + i * C, C)], sos[b])
        return carry

    lax.fori_loop(0, NCHUNK // 2, step, 0)
    wait_out(0)
    wait_out(1)


@jax.jit
def _run(x3d, xseg3d, tok513, pos_table, seg_table):
    tokpos = pl.pallas_call(
        _tokpos_tc_body,
        out_shape=jax.ShapeDtypeStruct((POS_ROWS, H), jnp.float32),
    )(tok513, pos_table)

    mesh = plsc.VectorSubcoreMesh(core_axis_name="c", subcore_axis_name="s")
    call = pl.kernel(
        _sc_body,
        out_type=jax.ShapeDtypeStruct((N, H), jnp.float32),
        mesh=mesh,
        scratch_types=[
            pltpu.VMEM((NCHUNK, C), jnp.int32),
            pltpu.VMEM((NCHUNK, C), jnp.int32),
            pltpu.VMEM((POS_ROWS, H), jnp.float32),
            pltpu.VMEM((SEG_ROWS, H), jnp.float32),
            pltpu.VMEM((C, H), jnp.float32),
            pltpu.VMEM((C, H), jnp.float32),
            pltpu.SemaphoreType.DMA,
            pltpu.SemaphoreType.DMA,
        ],
    )
    return call(x3d, xseg3d, tokpos, seg_table)


def kernel(x, x_seg, token_table, pos_table, seg_table):
    x3d = x.reshape(NW, NCHUNK, C)
    xseg3d = x_seg.reshape(NW, NCHUNK, C)
    out = _run(x3d, xseg3d, token_table[:POS_ROWS], pos_table, seg_table)
    return out.reshape(B, L, H)


# speedup vs baseline: 1.1541x; 1.1541x over previous
import functools

import jax
import jax.numpy as jnp
from jax import lax
from jax.experimental import pallas as pl
from jax.experimental.pallas import tpu as pltpu
from jax.experimental.pallas import tpu_sc as plsc

B = 1024
L = 200
H = 128
POS_ROWS = 513
SEG_ROWS = 3
N = B * L
NW = 32
PER_W = N // NW
C = 128
NCHUNK = PER_W // C
NCOL = H // 16


def _tokpos_tc_body(tok_ref, pos_ref, out_ref):
    out_ref[...] = tok_ref[...] + pos_ref[...]


def _sc_body(x_hbm, xseg_hbm, tokpos_hbm, seg_hbm, out_hbm,
             xi, si, tp, sg, st0, st1, so0, so1):
    wid = lax.axis_index("s") * 2 + lax.axis_index("c")
    base = wid * PER_W
    pltpu.sync_copy(x_hbm.at[wid], xi)
    pltpu.sync_copy(xseg_hbm.at[wid], si)
    pltpu.sync_copy(tokpos_hbm, tp)
    pltpu.sync_copy(seg_hbm, sg)

    segv = [[sg[s, pl.ds(j * 16, 16)] for j in range(NCOL)]
            for s in range(SEG_ROWS)]
    stages = (st0, st1)
    sos = (so0, so1)

    def wait_out(b):
        pltpu.make_async_copy(
            stages[b], out_hbm.at[pl.ds(base, C)], sos[b]).wait()

    def expand_chunk(i, b):
        st = stages[b]

        @plsc.parallel_loop(0, C // 16, unroll=2)
        def grp(g):
            pvec = xi[i, pl.ds(g * 16, 16)]
            svec = si[i, pl.ds(g * 16, 16)]
            for l in range(16):
                p = pvec[l]
                s = svec[l]
                m1 = s == 1
                m2 = s == 2
                for j in range(NCOL):
                    tv = tp[p, pl.ds(j * 16, 16)]
                    sv = jnp.where(m2, segv[2][j],
                                   jnp.where(m1, segv[1][j], segv[0][j]))
                    st[g * 16 + l, pl.ds(j * 16, 16)] = tv + sv

    def step(k, carry):
        for b in range(2):
            i = 2 * k + b

            @pl.when(k > 0)
            def _():
                wait_out(b)

            expand_chunk(i, b)
            pltpu.async_copy(
                stages[b], out_hbm.at[pl.ds(base + i * C, C)], sos[b])
        return carry

    lax.fori_loop(0, NCHUNK // 2, step, 0)
    wait_out(0)
    wait_out(1)


@jax.jit
def _run(x3d, xseg3d, tok513, pos_table, seg_table):
    tokpos = pl.pallas_call(
        _tokpos_tc_body,
        out_shape=jax.ShapeDtypeStruct((POS_ROWS, H), jnp.float32),
    )(tok513, pos_table)

    mesh = plsc.VectorSubcoreMesh(core_axis_name="c", subcore_axis_name="s")
    call = pl.kernel(
        _sc_body,
        out_type=jax.ShapeDtypeStruct((N, H), jnp.float32),
        mesh=mesh,
        scratch_types=[
            pltpu.VMEM((NCHUNK, C), jnp.int32),
            pltpu.VMEM((NCHUNK, C), jnp.int32),
            pltpu.VMEM((POS_ROWS, H), jnp.float32),
            pltpu.VMEM((SEG_ROWS, H), jnp.float32),
            pltpu.VMEM((C, H), jnp.float32),
            pltpu.VMEM((C, H), jnp.float32),
            pltpu.SemaphoreType.DMA,
            pltpu.SemaphoreType.DMA,
        ],
    )
    return call(x3d, xseg3d, tokpos, seg_table)


def kernel(x, x_seg, token_table, pos_table, seg_table):
    x3d = x.reshape(NW, NCHUNK, C)
    xseg3d = x_seg.reshape(NW, NCHUNK, C)
    out = _run(x3d, xseg3d, token_table[:POS_ROWS], pos_table, seg_table)
    return out.reshape(B, L, H)
